# baseline (speedup 1.0000x reference)
import functools

import jax
import jax.numpy as jnp
from jax import lax
from jax.experimental import pallas as pl
from jax.experimental.pallas import tpu as pltpu
from jax.experimental.pallas import tpu_sc as plsc

N = 10000
NPAD = 10240
E = 320000
ALPHA = 0.2
PPR_ITERS = 10
BM = 256
NBLK = NPAD // BM
RPT = NPAD // 16
DUMMY = 240
CH1 = 80
CH2 = 160
G = 16

_mesh = plsc.VectorSubcoreMesh(core_axis_name="c", subcore_axis_name="s")


def _make_sc_scatter(ch):
    @functools.partial(
        pl.kernel,
        mesh=_mesh,
        out_type=jax.ShapeDtypeStruct((2, NPAD, 128), jnp.float32),
        scratch_types=[
            pltpu.VMEM((G, 128), jnp.int32),
            pltpu.VMEM((G, 128), jnp.int32),
            pltpu.VMEM((128, 128), jnp.float32),
            pltpu.VMEM((128, 128), jnp.float32),
            pltpu.VMEM_SHARED((NPAD, 128), jnp.float32),
            pltpu.SemaphoreType.DMA,
            pltpu.SemaphoreType.DMA,
            pltpu.SemaphoreType.DMA,
            pltpu.SemaphoreType.DMA,
        ],
    )
    def sc_scatter(table, srcs, dsts, out, src_v, dst_v, rows_a, rows_b, acc,
                   gsem_a, gsem_b, ssem_a, ssem_b):
        c = lax.axis_index("c")
        s = lax.axis_index("s")
        rows = (rows_a, rows_b)
        gsems = (gsem_a, gsem_b)
        ssems = (ssem_a, ssem_b)

        zeros = jnp.zeros((16,), jnp.float32)

        @pl.loop(0, 128)
        def _zero_rows(i):
            for q in range(8):
                rows_a[i, pl.ds(q * 16, 16)] = zeros

        row0 = s * RPT
        for j in range(RPT // 128):
            pltpu.sync_copy(rows_a, acc.at[pl.ds(row0 + j * 128, 128)])
        plsc.subcore_barrier()

        @pl.loop(0, ch // G)
        def _group(g):
            pltpu.sync_copy(srcs.at[c, s, pl.ds(g * G, G)], src_v)
            pltpu.sync_copy(dsts.at[c, s, pl.ds(g * G, G)], dst_v)
            gd = {0: pltpu.async_copy(table.at[src_v.at[0]], rows[0],
                                      gsems[0])}
            sd = {}
            for k in range(G):
                b = k % 2
                nb = (k + 1) % 2
                if k >= 1:
                    sd[k - 1].wait()
                if k + 1 < G:
                    gd[k + 1] = pltpu.async_copy(
                        table.at[src_v.at[k + 1]], rows[nb], gsems[nb])
                gd[k].wait()
                sd[k] = pltpu.async_copy(rows[b], acc.at[dst_v.at[k]],
                                         ssems[b], add=True)
            sd[G - 1].wait()

        plsc.subcore_barrier()
        for j in range(RPT // 128):
            pltpu.sync_copy(acc.at[pl.ds(row0 + j * 128, 128)],
                            out.at[c, pl.ds(row0 + j * 128, 128)])

    return sc_scatter


_sc_scatter128 = _make_sc_scatter(CH1)
_sc_scatter256 = _make_sc_scatter(CH2)


@functools.partial(
    pl.kernel,
    mesh=_mesh,
    out_type=jax.ShapeDtypeStruct((2, NPAD), jnp.float32),
    scratch_types=[
        pltpu.VMEM((G, 128), jnp.int32),
        pltpu.VMEM((128,), jnp.float32),
        pltpu.VMEM((RPT,), jnp.float32),
        pltpu.VMEM_SHARED((NPAD,), jnp.float32),
        pltpu.SemaphoreType.DMA,
    ],
)
def _sc_deg(dsts, out, dst_v, ones_v, z_v, acc1, sem):
    c = lax.axis_index("c")
    s = lax.axis_index("s")
    ones = jnp.ones((16,), jnp.float32)
    zeros = jnp.zeros((16,), jnp.float32)
    for q in range(8):
        ones_v[pl.ds(q * 16, 16)] = ones
    for q in range(RPT // 16):
        z_v[pl.ds(q * 16, 16)] = zeros
    row0 = s * RPT
    pltpu.sync_copy(z_v, acc1.at[pl.ds(row0, RPT)])
    plsc.subcore_barrier()

    @pl.loop(0, CH1 // G)
    def _group(g):
        pltpu.sync_copy(dsts.at[c, s, pl.ds(g * G, G)], dst_v)
        descs = [pltpu.async_copy(ones_v, acc1.at[dst_v.at[k]], sem, add=True)
                 for k in range(G)]
        for d in descs:
            d.wait()

    plsc.subcore_barrier()
    pltpu.sync_copy(acc1.at[pl.ds(row0, RPT)], out.at[c, pl.ds(row0, RPT)])


def _dis_body(dpa_ref, dpb_ref, x_ref, disb_ref, zt0_ref):
    deg = dpa_ref[...][0] + dpb_ref[...][0] + 1.0
    db = jnp.broadcast_to(lax.rsqrt(deg), (BM, 128))
    disb_ref[...] = db
    zt0_ref[...] = db * x_ref[...]


def _tc_dis(dp, x):
    return pl.pallas_call(
        _dis_body,
        grid=(NBLK,),
        in_specs=[
            pl.BlockSpec((1, BM, 1), lambda i: (0, i, 0)),
            pl.BlockSpec((1, BM, 1), lambda i: (1, i, 0)),
            pl.BlockSpec((BM, 128), lambda i: (i, 0)),
        ],
        out_specs=[
            pl.BlockSpec((BM, 128), lambda i: (i, 0)),
            pl.BlockSpec((BM, 128), lambda i: (i, 0)),
        ],
        out_shape=[
            jax.ShapeDtypeStruct((NPAD, 128), jnp.float32),
            jax.ShapeDtypeStruct((NPAD, 128), jnp.float32),
        ],
    )(dp, dp, x)


def _ppr_body(x_ref, zt_ref, p0_ref, p1_ref, db_ref, z_ref, ztn_ref):
    db = db_ref[...]
    t = p0_ref[...][0] + p1_ref[...][0] + zt_ref[...]
    znew = ALPHA * x_ref[...] + (1.0 - ALPHA) * (db * t)
    z_ref[...] = znew
    ztn_ref[...] = db * znew


def _tc_ppr(x, zt, p, disb):
    return pl.pallas_call(
        _ppr_body,
        grid=(NBLK,),
        in_specs=[
            pl.BlockSpec((BM, 128), lambda i: (i, 0)),
            pl.BlockSpec((BM, 128), lambda i: (i, 0)),
            pl.BlockSpec((1, BM, 128), lambda i: (0, i, 0)),
            pl.BlockSpec((1, BM, 128), lambda i: (1, i, 0)),
            pl.BlockSpec((BM, 128), lambda i: (i, 0)),
        ],
        out_specs=[
            pl.BlockSpec((BM, 128), lambda i: (i, 0)),
            pl.BlockSpec((BM, 128), lambda i: (i, 0)),
        ],
        out_shape=[
            jax.ShapeDtypeStruct((NPAD, 128), jnp.float32),
            jax.ShapeDtypeStruct((NPAD, 128), jnp.float32),
        ],
    )(x, zt, p, p, disb)


def _mm_body(x_ref, w_ref, db_ref, out_ref):
    out_ref[...] = db_ref[...] * jnp.dot(
        x_ref[...], w_ref[...], preferred_element_type=jnp.float32)


def _tc_mm_scale(x, w, disb):
    k = x.shape[1]
    return pl.pallas_call(
        _mm_body,
        grid=(NBLK, 2),
        in_specs=[
            pl.BlockSpec((BM, k), lambda i, c: (i, 0)),
            pl.BlockSpec((k, 128), lambda i, c: (0, c)),
            pl.BlockSpec((BM, 128), lambda i, c: (i, 0)),
        ],
        out_specs=pl.BlockSpec((BM, 128), lambda i, c: (c * NBLK + i, 0)),
        out_shape=jax.ShapeDtypeStruct((2 * NPAD, 128), jnp.float32),
    )(x, w, disb)


def _fuse_body(s0_ref, s1_ref, at0_ref, at1_ref, w_ref, db_ref, out_ref):
    db = db_ref[...]
    h0 = jnp.maximum(db * (s0_ref[...][0] + at0_ref[...]), 0.0)
    h1 = jnp.maximum(db * (s1_ref[...][0] + at1_ref[...]), 0.0)
    h = jnp.concatenate([h0, h1], axis=1)
    out_ref[...] = db * jnp.dot(h, w_ref[...],
                                preferred_element_type=jnp.float32)


def _tc_fuse_mm(s, at, w, disb):
    return pl.pallas_call(
        _fuse_body,
        grid=(NBLK, 2),
        in_specs=[
            pl.BlockSpec((1, BM, 128), lambda i, c: (0, i, 0)),
            pl.BlockSpec((1, BM, 128), lambda i, c: (1, i, 0)),
            pl.BlockSpec((BM, 128), lambda i, c: (i, 0)),
            pl.BlockSpec((BM, 128), lambda i, c: (NBLK + i, 0)),
            pl.BlockSpec((256, 128), lambda i, c: (0, c)),
            pl.BlockSpec((BM, 128), lambda i, c: (i, 0)),
        ],
        out_specs=pl.BlockSpec((BM, 128), lambda i, c: (c * NBLK + i, 0)),
        out_shape=jax.ShapeDtypeStruct((2 * NPAD, 128), jnp.float32),
    )(s, s, at, at, w, disb)


def _final_body(s2a_ref, s2b_ref, b1a_ref, b1b_ref, s4a_ref, s4b_ref,
                b2a_ref, b2b_ref, db_ref, z1_ref, z3_ref, z_ref):
    db = db_ref[...]
    z1a = jnp.maximum(db * (s2a_ref[...][0] + b1a_ref[...]), 0.0)
    z1b = jnp.maximum(db * (s2b_ref[...][0] + b1b_ref[...]), 0.0)
    z3a = jnp.maximum(db * (s4a_ref[...][0] + b2a_ref[...]), 0.0)
    z3b = jnp.maximum(db * (s4b_ref[...][0] + b2b_ref[...]), 0.0)
    z1 = jnp.concatenate([z1a, z1b], axis=1)
    z3 = jnp.concatenate([z3a, z3b], axis=1)
    n2 = (2.0 * jnp.sum(z1 * z1, axis=1, keepdims=True)
          + jnp.sum(z3 * z3, axis=1, keepdims=True))
    inv = 1.0 / (jnp.sqrt(n2) + 1e-12)
    z1_ref[...] = z1
    z3_ref[...] = z3
    z_ref[...] = jnp.concatenate([z1, z1, z3], axis=1) * inv


def _tc_final(s2, b1t, s4, b2t, disb):
    half = lambda j: pl.BlockSpec((1, BM, 128), lambda i, j=j: (j, i, 0))
    flat = lambda j: pl.BlockSpec((BM, 128), lambda i, j=j: (j * NBLK + i, 0))
    return pl.pallas_call(
        _final_body,
        grid=(NBLK,),
        in_specs=[half(0), half(1), flat(0), flat(1),
                  half(0), half(1), flat(0), flat(1),
                  pl.BlockSpec((BM, 128), lambda i: (i, 0))],
        out_specs=[
            pl.BlockSpec((BM, 256), lambda i: (i, 0)),
            pl.BlockSpec((BM, 256), lambda i: (i, 0)),
            pl.BlockSpec((BM, 768), lambda i: (i, 0)),
        ],
        out_shape=[
            jax.ShapeDtypeStruct((NPAD, 256), jnp.float32),
            jax.ShapeDtypeStruct((NPAD, 256), jnp.float32),
            jax.ShapeDtypeStruct((NPAD, 768), jnp.float32),
        ],
    )(s2, s2, b1t, b1t, s4, s4, b2t, b2t, disb)


def _edge_layout(src, dst):
    i32 = jnp.int32
    order = jnp.argsort(src)
    src = src[order]
    dst = dst[order]
    p1 = 2 * 16 * CH1 * 128 - E
    pad_s = (jnp.arange(p1, dtype=i32) * 97) % N
    pad_d = N + jnp.arange(p1, dtype=i32) % DUMMY
    src1 = jnp.concatenate([src, pad_s]).reshape(2, 16, CH1, 128)
    dst1 = jnp.concatenate([dst, pad_d]).reshape(2, 16, CH1, 128)
    p2 = 16 * CH2 * 128 - E
    ps2 = (jnp.arange(p2, dtype=i32) * 97) % N
    pd2 = N + jnp.arange(p2, dtype=i32) % DUMMY
    srcf = jnp.concatenate([src, ps2]).reshape(16, CH2, 128)
    dstf = jnp.concatenate([dst, pd2]).reshape(16, CH2, 128)
    src2 = jnp.stack([srcf, srcf + NPAD])
    dst2 = jnp.stack([dstf, dstf])
    return src1, dst1, src2, dst2


def kernel(x_feature, llmfeatures, W1a, W1b, W2a, W2b, Wt1, Wt2, Wp1, Wp2,
           Wm1, Wm2, edge_index):
    x = jnp.pad(x_feature, ((0, NPAD - N), (0, 0)))
    llm = jnp.pad(llmfeatures, ((0, NPAD - N), (0, 0)))
    src1, dst1, src2, dst2 = _edge_layout(edge_index[0], edge_index[1])

    degp = _sc_deg(dst1)
    dp = degp.reshape(2, NPAD, 1)
    disb, zt = _tc_dis(dp, x)
    z = x
    for _ in range(PPR_ITERS):
        p = _sc_scatter128(zt, src1, dst1)
        z, zt = _tc_ppr(x, zt, p, disb)

    a1t = _tc_mm_scale(z, W1a, disb)
    s1 = _sc_scatter256(a1t, src2, dst2)
    b1t = _tc_fuse_mm(s1, a1t, W1b, disb)
    s2 = _sc_scatter256(b1t, src2, dst2)

    a2t = _tc_mm_scale(llm, W2a, disb)
    s3 = _sc_scatter256(a2t, src2, dst2)
    b2t = _tc_fuse_mm(s3, a2t, W2b, disb)
    s4 = _sc_scatter256(b2t, src2, dst2)

    z1, z3, zz = _tc_final(s2, b1t, s4, b2t, disb)
    return (z3[:N], zz[:N], z1[:N], z1[:N])


# speedup vs baseline: 2.0661x; 2.0661x over previous
import functools

import jax
import jax.numpy as jnp
from jax import lax
from jax.experimental import pallas as pl
from jax.experimental.pallas import tpu as pltpu
from jax.experimental.pallas import tpu_sc as plsc

N = 10000
NPAD = 10240
E = 320000
ALPHA = 0.2
PPR_ITERS = 10
BM = 256
NBLK = NPAD // BM
RPT = NPAD // 16
DUMMY = 240
CH1 = 80
CH2 = 160
G = 16

_mesh = plsc.VectorSubcoreMesh(core_axis_name="c", subcore_axis_name="s")


def _make_sc_scatter(ch):
    @functools.partial(
        pl.kernel,
        mesh=_mesh,
        out_type=jax.ShapeDtypeStruct((2, NPAD, 128), jnp.float32),
        scratch_types=[
            pltpu.VMEM((G, 128), jnp.int32),
            pltpu.VMEM((G, 128), jnp.int32),
            pltpu.VMEM((128, 128), jnp.float32),
            pltpu.VMEM((128, 128), jnp.float32),
            pltpu.VMEM_SHARED((NPAD, 128), jnp.float32),
            pltpu.SemaphoreType.DMA,
            pltpu.SemaphoreType.DMA,
            pltpu.SemaphoreType.DMA,
            pltpu.SemaphoreType.DMA,
        ],
    )
    def sc_scatter(table, srcs, dsts, out, src_v, dst_v, rows_a, rows_b, acc,
                   gsem_a, gsem_b, ssem_a, ssem_b):
        c = lax.axis_index("c")
        s = lax.axis_index("s")
        rows = (rows_a, rows_b)
        gsems = (gsem_a, gsem_b)
        ssems = (ssem_a, ssem_b)

        zeros = jnp.zeros((16,), jnp.float32)

        @pl.loop(0, 128)
        def _zero_rows(i):
            for q in range(8):
                rows_a[i, pl.ds(q * 16, 16)] = zeros

        row0 = s * RPT
        for j in range(RPT // 128):
            pltpu.sync_copy(rows_a, acc.at[pl.ds(row0 + j * 128, 128)])
        plsc.subcore_barrier()

        @pl.loop(0, ch // G)
        def _group(g):
            pltpu.sync_copy(srcs.at[c, s, pl.ds(g * G, G)], src_v)
            pltpu.sync_copy(dsts.at[c, s, pl.ds(g * G, G)], dst_v)
            gd = {0: pltpu.async_copy(table.at[src_v.at[0]], rows[0],
                                      gsems[0])}
            sd = {}
            for k in range(G):
                b = k % 2
                nb = (k + 1) % 2
                if k >= 1:
                    sd[k - 1].wait()
                if k + 1 < G:
                    gd[k + 1] = pltpu.async_copy(
                        table.at[src_v.at[k + 1]], rows[nb], gsems[nb])
                gd[k].wait()
                sd[k] = pltpu.async_copy(rows[b], acc.at[dst_v.at[k]],
                                         ssems[b], add=True)
            sd[G - 1].wait()

        plsc.subcore_barrier()
        for j in range(RPT // 128):
            pltpu.sync_copy(acc.at[pl.ds(row0 + j * 128, 128)],
                            out.at[c, pl.ds(row0 + j * 128, 128)])

    return sc_scatter


_sc_scatter128 = _make_sc_scatter(CH1)
_sc_scatter256 = _make_sc_scatter(CH2)


@functools.partial(
    pl.kernel,
    mesh=_mesh,
    out_type=jax.ShapeDtypeStruct((2, NPAD), jnp.float32),
    scratch_types=[
        pltpu.VMEM((G, 128), jnp.int32),
        pltpu.VMEM((128,), jnp.float32),
        pltpu.VMEM((RPT,), jnp.float32),
        pltpu.VMEM_SHARED((NPAD,), jnp.float32),
        pltpu.SemaphoreType.DMA,
    ],
)
def _sc_deg(dsts, out, dst_v, ones_v, z_v, acc1, sem):
    c = lax.axis_index("c")
    s = lax.axis_index("s")
    ones = jnp.ones((16,), jnp.float32)
    zeros = jnp.zeros((16,), jnp.float32)
    for q in range(8):
        ones_v[pl.ds(q * 16, 16)] = ones
    for q in range(RPT // 16):
        z_v[pl.ds(q * 16, 16)] = zeros
    row0 = s * RPT
    pltpu.sync_copy(z_v, acc1.at[pl.ds(row0, RPT)])
    plsc.subcore_barrier()

    @pl.loop(0, CH1 // G)
    def _group(g):
        pltpu.sync_copy(dsts.at[c, s, pl.ds(g * G, G)], dst_v)
        descs = [pltpu.async_copy(ones_v, acc1.at[dst_v.at[k]], sem, add=True)
                 for k in range(G)]
        for d in descs:
            d.wait()

    plsc.subcore_barrier()
    pltpu.sync_copy(acc1.at[pl.ds(row0, RPT)], out.at[c, pl.ds(row0, RPT)])


def _dis_body(dpa_ref, dpb_ref, x_ref, disb_ref, zt0_ref):
    deg = dpa_ref[...][0] + dpb_ref[...][0] + 1.0
    db = jnp.broadcast_to(lax.rsqrt(deg), (BM, 128))
    disb_ref[...] = db
    zt0_ref[...] = db * x_ref[...]


def _tc_dis(dp, x):
    return pl.pallas_call(
        _dis_body,
        grid=(NBLK,),
        in_specs=[
            pl.BlockSpec((1, BM, 1), lambda i: (0, i, 0)),
            pl.BlockSpec((1, BM, 1), lambda i: (1, i, 0)),
            pl.BlockSpec((BM, 128), lambda i: (i, 0)),
        ],
        out_specs=[
            pl.BlockSpec((BM, 128), lambda i: (i, 0)),
            pl.BlockSpec((BM, 128), lambda i: (i, 0)),
        ],
        out_shape=[
            jax.ShapeDtypeStruct((NPAD, 128), jnp.float32),
            jax.ShapeDtypeStruct((NPAD, 128), jnp.float32),
        ],
    )(dp, dp, x)


def _ppr_body(x_ref, zt_ref, p0_ref, p1_ref, db_ref, z_ref, ztn_ref):
    db = db_ref[...]
    t = p0_ref[...][0] + p1_ref[...][0] + zt_ref[...]
    znew = ALPHA * x_ref[...] + (1.0 - ALPHA) * (db * t)
    z_ref[...] = znew
    ztn_ref[...] = db * znew


def _tc_ppr(x, zt, p, disb):
    return pl.pallas_call(
        _ppr_body,
        grid=(NBLK,),
        in_specs=[
            pl.BlockSpec((BM, 128), lambda i: (i, 0)),
            pl.BlockSpec((BM, 128), lambda i: (i, 0)),
            pl.BlockSpec((1, BM, 128), lambda i: (0, i, 0)),
            pl.BlockSpec((1, BM, 128), lambda i: (1, i, 0)),
            pl.BlockSpec((BM, 128), lambda i: (i, 0)),
        ],
        out_specs=[
            pl.BlockSpec((BM, 128), lambda i: (i, 0)),
            pl.BlockSpec((BM, 128), lambda i: (i, 0)),
        ],
        out_shape=[
            jax.ShapeDtypeStruct((NPAD, 128), jnp.float32),
            jax.ShapeDtypeStruct((NPAD, 128), jnp.float32),
        ],
    )(x, zt, p, p, disb)


def _mm_body(x_ref, w_ref, db_ref, out_ref):
    out_ref[...] = db_ref[...] * jnp.dot(
        x_ref[...], w_ref[...], preferred_element_type=jnp.float32)


def _tc_mm_scale(x, w, disb):
    k = x.shape[1]
    return pl.pallas_call(
        _mm_body,
        grid=(NBLK, 2),
        in_specs=[
            pl.BlockSpec((BM, k), lambda i, c: (i, 0)),
            pl.BlockSpec((k, 128), lambda i, c: (0, c)),
            pl.BlockSpec((BM, 128), lambda i, c: (i, 0)),
        ],
        out_specs=pl.BlockSpec((BM, 128), lambda i, c: (c * NBLK + i, 0)),
        out_shape=jax.ShapeDtypeStruct((2 * NPAD, 128), jnp.float32),
    )(x, w, disb)


def _fuse_body(s0_ref, s1_ref, at0_ref, at1_ref, w_ref, db_ref, out_ref):
    db = db_ref[...]
    h0 = jnp.maximum(db * (s0_ref[...][0] + at0_ref[...]), 0.0)
    h1 = jnp.maximum(db * (s1_ref[...][0] + at1_ref[...]), 0.0)
    h = jnp.concatenate([h0, h1], axis=1)
    out_ref[...] = db * jnp.dot(h, w_ref[...],
                                preferred_element_type=jnp.float32)


def _tc_fuse_mm(s, at, w, disb):
    return pl.pallas_call(
        _fuse_body,
        grid=(NBLK, 2),
        in_specs=[
            pl.BlockSpec((1, BM, 128), lambda i, c: (0, i, 0)),
            pl.BlockSpec((1, BM, 128), lambda i, c: (1, i, 0)),
            pl.BlockSpec((BM, 128), lambda i, c: (i, 0)),
            pl.BlockSpec((BM, 128), lambda i, c: (NBLK + i, 0)),
            pl.BlockSpec((256, 128), lambda i, c: (0, c)),
            pl.BlockSpec((BM, 128), lambda i, c: (i, 0)),
        ],
        out_specs=pl.BlockSpec((BM, 128), lambda i, c: (c * NBLK + i, 0)),
        out_shape=jax.ShapeDtypeStruct((2 * NPAD, 128), jnp.float32),
    )(s, s, at, at, w, disb)


def _final_body(s2a_ref, s2b_ref, b1a_ref, b1b_ref, s4a_ref, s4b_ref,
                b2a_ref, b2b_ref, db_ref, z1_ref, z3_ref, z_ref):
    db = db_ref[...]
    z1a = jnp.maximum(db * (s2a_ref[...][0] + b1a_ref[...]), 0.0)
    z1b = jnp.maximum(db * (s2b_ref[...][0] + b1b_ref[...]), 0.0)
    z3a = jnp.maximum(db * (s4a_ref[...][0] + b2a_ref[...]), 0.0)
    z3b = jnp.maximum(db * (s4b_ref[...][0] + b2b_ref[...]), 0.0)
    z1 = jnp.concatenate([z1a, z1b], axis=1)
    z3 = jnp.concatenate([z3a, z3b], axis=1)
    n2 = (2.0 * jnp.sum(z1 * z1, axis=1, keepdims=True)
          + jnp.sum(z3 * z3, axis=1, keepdims=True))
    inv = 1.0 / (jnp.sqrt(n2) + 1e-12)
    z1_ref[...] = z1
    z3_ref[...] = z3
    z_ref[...] = jnp.concatenate([z1, z1, z3], axis=1) * inv


def _tc_final(s2, b1t, s4, b2t, disb):
    half = lambda j: pl.BlockSpec((1, BM, 128), lambda i, j=j: (j, i, 0))
    flat = lambda j: pl.BlockSpec((BM, 128), lambda i, j=j: (j * NBLK + i, 0))
    return pl.pallas_call(
        _final_body,
        grid=(NBLK,),
        in_specs=[half(0), half(1), flat(0), flat(1),
                  half(0), half(1), flat(0), flat(1),
                  pl.BlockSpec((BM, 128), lambda i: (i, 0))],
        out_specs=[
            pl.BlockSpec((BM, 256), lambda i: (i, 0)),
            pl.BlockSpec((BM, 256), lambda i: (i, 0)),
            pl.BlockSpec((BM, 768), lambda i: (i, 0)),
        ],
        out_shape=[
            jax.ShapeDtypeStruct((NPAD, 256), jnp.float32),
            jax.ShapeDtypeStruct((NPAD, 256), jnp.float32),
            jax.ShapeDtypeStruct((NPAD, 768), jnp.float32),
        ],
    )(s2, s2, b1t, b1t, s4, s4, b2t, b2t, disb)


def _edge_layout(src, dst):
    i32 = jnp.int32
    p1 = 2 * 16 * CH1 * 128 - E
    pad_s = (jnp.arange(p1, dtype=i32) * 97) % N
    pad_d = N + jnp.arange(p1, dtype=i32) % DUMMY
    src1 = jnp.concatenate([src, pad_s]).reshape(2, 16, CH1, 128)
    dst1 = jnp.concatenate([dst, pad_d]).reshape(2, 16, CH1, 128)
    p2 = 16 * CH2 * 128 - E
    ps2 = (jnp.arange(p2, dtype=i32) * 97) % N
    pd2 = N + jnp.arange(p2, dtype=i32) % DUMMY
    srcf = jnp.concatenate([src, ps2]).reshape(16, CH2, 128)
    dstf = jnp.concatenate([dst, pd2]).reshape(16, CH2, 128)
    src2 = jnp.stack([srcf, srcf + NPAD])
    dst2 = jnp.stack([dstf, dstf])
    return src1, dst1, src2, dst2


def kernel(x_feature, llmfeatures, W1a, W1b, W2a, W2b, Wt1, Wt2, Wp1, Wp2,
           Wm1, Wm2, edge_index):
    x = jnp.pad(x_feature, ((0, NPAD - N), (0, 0)))
    llm = jnp.pad(llmfeatures, ((0, NPAD - N), (0, 0)))
    src1, dst1, src2, dst2 = _edge_layout(edge_index[0], edge_index[1])

    degp = _sc_deg(dst1)
    dp = degp.reshape(2, NPAD, 1)
    disb, zt = _tc_dis(dp, x)
    z = x
    for _ in range(PPR_ITERS):
        p = _sc_scatter128(zt, src1, dst1)
        z, zt = _tc_ppr(x, zt, p, disb)

    a1t = _tc_mm_scale(z, W1a, disb)
    s1 = _sc_scatter256(a1t, src2, dst2)
    b1t = _tc_fuse_mm(s1, a1t, W1b, disb)
    s2 = _sc_scatter256(b1t, src2, dst2)

    a2t = _tc_mm_scale(llm, W2a, disb)
    s3 = _sc_scatter256(a2t, src2, dst2)
    b2t = _tc_fuse_mm(s3, a2t, W2b, disb)
    s4 = _sc_scatter256(b2t, src2, dst2)

    z1, z3, zz = _tc_final(s2, b1t, s4, b2t, disb)
    return (z3[:N], zz[:N], z1[:N], z1[:N])


# speedup vs baseline: 2.3598x; 1.1422x over previous
import functools

import jax
import jax.numpy as jnp
from jax import lax
from jax.experimental import pallas as pl
from jax.experimental.pallas import tpu as pltpu
from jax.experimental.pallas import tpu_sc as plsc

N = 10000
NPAD = 10240
E = 320000
ALPHA = 0.2
PPR_ITERS = 10
BM = 256
NBLK = NPAD // BM
RPT = NPAD // 16
DUMMY = 240
CH1 = 80
CH2 = 160
G = 32

_mesh = plsc.VectorSubcoreMesh(core_axis_name="c", subcore_axis_name="s")


def _make_sc_scatter(ch):
    @functools.partial(
        pl.kernel,
        mesh=_mesh,
        out_type=jax.ShapeDtypeStruct((2, NPAD, 128), jnp.float32),
        scratch_types=[
            pltpu.VMEM((G, 128), jnp.int32),
            pltpu.VMEM((G, 128), jnp.int32),
            pltpu.VMEM((128, 128), jnp.float32),
            pltpu.VMEM((128, 128), jnp.float32),
            pltpu.VMEM_SHARED((NPAD, 128), jnp.float32),
            pltpu.SemaphoreType.DMA,
            pltpu.SemaphoreType.DMA,
            pltpu.SemaphoreType.DMA,
            pltpu.SemaphoreType.DMA,
        ],
    )
    def sc_scatter(table, srcs, dsts, out, src_v, dst_v, rows_a, rows_b, acc,
                   gsem_a, gsem_b, ssem_a, ssem_b):
        c = lax.axis_index("c")
        s = lax.axis_index("s")
        rows = (rows_a, rows_b)
        gsems = (gsem_a, gsem_b)
        ssems = (ssem_a, ssem_b)

        zeros = jnp.zeros((16,), jnp.float32)

        @pl.loop(0, 128)
        def _zero_rows(i):
            for q in range(8):
                rows_a[i, pl.ds(q * 16, 16)] = zeros

        row0 = s * RPT
        for j in range(RPT // 128):
            pltpu.sync_copy(rows_a, acc.at[pl.ds(row0 + j * 128, 128)])
        plsc.subcore_barrier()

        @pl.loop(0, ch // G)
        def _group(g):
            pltpu.sync_copy(srcs.at[c, s, pl.ds(g * G, G)], src_v)
            pltpu.sync_copy(dsts.at[c, s, pl.ds(g * G, G)], dst_v)
            gd = {0: pltpu.async_copy(table.at[src_v.at[0]], rows[0],
                                      gsems[0])}
            sd = {}
            for k in range(G):
                b = k % 2
                nb = (k + 1) % 2
                if k >= 1:
                    sd[k - 1].wait()
                if k + 1 < G:
                    gd[k + 1] = pltpu.async_copy(
                        table.at[src_v.at[k + 1]], rows[nb], gsems[nb])
                gd[k].wait()
                sd[k] = pltpu.async_copy(rows[b], acc.at[dst_v.at[k]],
                                         ssems[b], add=True)
            sd[G - 1].wait()

        plsc.subcore_barrier()
        for j in range(RPT // 128):
            pltpu.sync_copy(acc.at[pl.ds(row0 + j * 128, 128)],
                            out.at[c, pl.ds(row0 + j * 128, 128)])

    return sc_scatter


_sc_scatter128 = _make_sc_scatter(CH1)
_sc_scatter256 = _make_sc_scatter(CH2)


@functools.partial(
    pl.kernel,
    mesh=_mesh,
    out_type=jax.ShapeDtypeStruct((2, NPAD), jnp.float32),
    scratch_types=[
        pltpu.VMEM((G, 128), jnp.int32),
        pltpu.VMEM((128,), jnp.float32),
        pltpu.VMEM((RPT,), jnp.float32),
        pltpu.VMEM_SHARED((NPAD,), jnp.float32),
        pltpu.SemaphoreType.DMA,
    ],
)
def _sc_deg(dsts, out, dst_v, ones_v, z_v, acc1, sem):
    c = lax.axis_index("c")
    s = lax.axis_index("s")
    ones = jnp.ones((16,), jnp.float32)
    zeros = jnp.zeros((16,), jnp.float32)
    for q in range(8):
        ones_v[pl.ds(q * 16, 16)] = ones
    for q in range(RPT // 16):
        z_v[pl.ds(q * 16, 16)] = zeros
    row0 = s * RPT
    pltpu.sync_copy(z_v, acc1.at[pl.ds(row0, RPT)])
    plsc.subcore_barrier()

    @pl.loop(0, CH1 // G)
    def _group(g):
        pltpu.sync_copy(dsts.at[c, s, pl.ds(g * G, G)], dst_v)
        descs = [pltpu.async_copy(ones_v, acc1.at[dst_v.at[k]], sem, add=True)
                 for k in range(G)]
        for d in descs:
            d.wait()

    plsc.subcore_barrier()
    pltpu.sync_copy(acc1.at[pl.ds(row0, RPT)], out.at[c, pl.ds(row0, RPT)])


def _dis_body(dpa_ref, dpb_ref, x_ref, disb_ref, zt0_ref):
    deg = dpa_ref[...][0] + dpb_ref[...][0] + 1.0
    db = jnp.broadcast_to(lax.rsqrt(deg), (BM, 128))
    disb_ref[...] = db
    zt0_ref[...] = db * x_ref[...]


def _tc_dis(dp, x):
    return pl.pallas_call(
        _dis_body,
        grid=(NBLK,),
        in_specs=[
            pl.BlockSpec((1, BM, 1), lambda i: (0, i, 0)),
            pl.BlockSpec((1, BM, 1), lambda i: (1, i, 0)),
            pl.BlockSpec((BM, 128), lambda i: (i, 0)),
        ],
        out_specs=[
            pl.BlockSpec((BM, 128), lambda i: (i, 0)),
            pl.BlockSpec((BM, 128), lambda i: (i, 0)),
        ],
        out_shape=[
            jax.ShapeDtypeStruct((NPAD, 128), jnp.float32),
            jax.ShapeDtypeStruct((NPAD, 128), jnp.float32),
        ],
    )(dp, dp, x)


def _ppr_body(x_ref, zt_ref, p0_ref, p1_ref, db_ref, z_ref, ztn_ref):
    db = db_ref[...]
    t = p0_ref[...][0] + p1_ref[...][0] + zt_ref[...]
    znew = ALPHA * x_ref[...] + (1.0 - ALPHA) * (db * t)
    z_ref[...] = znew
    ztn_ref[...] = db * znew


def _tc_ppr(x, zt, p, disb):
    return pl.pallas_call(
        _ppr_body,
        grid=(NBLK,),
        in_specs=[
            pl.BlockSpec((BM, 128), lambda i: (i, 0)),
            pl.BlockSpec((BM, 128), lambda i: (i, 0)),
            pl.BlockSpec((1, BM, 128), lambda i: (0, i, 0)),
            pl.BlockSpec((1, BM, 128), lambda i: (1, i, 0)),
            pl.BlockSpec((BM, 128), lambda i: (i, 0)),
        ],
        out_specs=[
            pl.BlockSpec((BM, 128), lambda i: (i, 0)),
            pl.BlockSpec((BM, 128), lambda i: (i, 0)),
        ],
        out_shape=[
            jax.ShapeDtypeStruct((NPAD, 128), jnp.float32),
            jax.ShapeDtypeStruct((NPAD, 128), jnp.float32),
        ],
    )(x, zt, p, p, disb)


def _mm_body(x_ref, w_ref, db_ref, out_ref):
    out_ref[...] = db_ref[...] * jnp.dot(
        x_ref[...], w_ref[...], preferred_element_type=jnp.float32)


def _tc_mm_scale(x, w, disb):
    k = x.shape[1]
    return pl.pallas_call(
        _mm_body,
        grid=(NBLK, 2),
        in_specs=[
            pl.BlockSpec((BM, k), lambda i, c: (i, 0)),
            pl.BlockSpec((k, 128), lambda i, c: (0, c)),
            pl.BlockSpec((BM, 128), lambda i, c: (i, 0)),
        ],
        out_specs=pl.BlockSpec((BM, 128), lambda i, c: (c * NBLK + i, 0)),
        out_shape=jax.ShapeDtypeStruct((2 * NPAD, 128), jnp.float32),
    )(x, w, disb)


def _fuse_body(s0_ref, s1_ref, at0_ref, at1_ref, w_ref, db_ref, out_ref):
    db = db_ref[...]
    h0 = jnp.maximum(db * (s0_ref[...][0] + at0_ref[...]), 0.0)
    h1 = jnp.maximum(db * (s1_ref[...][0] + at1_ref[...]), 0.0)
    h = jnp.concatenate([h0, h1], axis=1)
    out_ref[...] = db * jnp.dot(h, w_ref[...],
                                preferred_element_type=jnp.float32)


def _tc_fuse_mm(s, at, w, disb):
    return pl.pallas_call(
        _fuse_body,
        grid=(NBLK, 2),
        in_specs=[
            pl.BlockSpec((1, BM, 128), lambda i, c: (0, i, 0)),
            pl.BlockSpec((1, BM, 128), lambda i, c: (1, i, 0)),
            pl.BlockSpec((BM, 128), lambda i, c: (i, 0)),
            pl.BlockSpec((BM, 128), lambda i, c: (NBLK + i, 0)),
            pl.BlockSpec((256, 128), lambda i, c: (0, c)),
            pl.BlockSpec((BM, 128), lambda i, c: (i, 0)),
        ],
        out_specs=pl.BlockSpec((BM, 128), lambda i, c: (c * NBLK + i, 0)),
        out_shape=jax.ShapeDtypeStruct((2 * NPAD, 128), jnp.float32),
    )(s, s, at, at, w, disb)


def _final_body(s2a_ref, s2b_ref, b1a_ref, b1b_ref, s4a_ref, s4b_ref,
                b2a_ref, b2b_ref, db_ref, z1_ref, z3_ref, z_ref):
    db = db_ref[...]
    z1a = jnp.maximum(db * (s2a_ref[...][0] + b1a_ref[...]), 0.0)
    z1b = jnp.maximum(db * (s2b_ref[...][0] + b1b_ref[...]), 0.0)
    z3a = jnp.maximum(db * (s4a_ref[...][0] + b2a_ref[...]), 0.0)
    z3b = jnp.maximum(db * (s4b_ref[...][0] + b2b_ref[...]), 0.0)
    z1 = jnp.concatenate([z1a, z1b], axis=1)
    z3 = jnp.concatenate([z3a, z3b], axis=1)
    n2 = (2.0 * jnp.sum(z1 * z1, axis=1, keepdims=True)
          + jnp.sum(z3 * z3, axis=1, keepdims=True))
    inv = 1.0 / (jnp.sqrt(n2) + 1e-12)
    z1_ref[...] = z1
    z3_ref[...] = z3
    z_ref[...] = jnp.concatenate([z1, z1, z3], axis=1) * inv


def _tc_final(s2, b1t, s4, b2t, disb):
    half = lambda j: pl.BlockSpec((1, BM, 128), lambda i, j=j: (j, i, 0))
    flat = lambda j: pl.BlockSpec((BM, 128), lambda i, j=j: (j * NBLK + i, 0))
    return pl.pallas_call(
        _final_body,
        grid=(NBLK,),
        in_specs=[half(0), half(1), flat(0), flat(1),
                  half(0), half(1), flat(0), flat(1),
                  pl.BlockSpec((BM, 128), lambda i: (i, 0))],
        out_specs=[
            pl.BlockSpec((BM, 256), lambda i: (i, 0)),
            pl.BlockSpec((BM, 256), lambda i: (i, 0)),
            pl.BlockSpec((BM, 768), lambda i: (i, 0)),
        ],
        out_shape=[
            jax.ShapeDtypeStruct((NPAD, 256), jnp.float32),
            jax.ShapeDtypeStruct((NPAD, 256), jnp.float32),
            jax.ShapeDtypeStruct((NPAD, 768), jnp.float32),
        ],
    )(s2, s2, b1t, b1t, s4, s4, b2t, b2t, disb)


def _edge_layout(src, dst):
    i32 = jnp.int32
    p1 = 2 * 16 * CH1 * 128 - E
    pad_s = (jnp.arange(p1, dtype=i32) * 97) % N
    pad_d = N + jnp.arange(p1, dtype=i32) % DUMMY
    src1 = jnp.concatenate([src, pad_s]).reshape(2, 16, CH1, 128)
    dst1 = jnp.concatenate([dst, pad_d]).reshape(2, 16, CH1, 128)
    p2 = 16 * CH2 * 128 - E
    ps2 = (jnp.arange(p2, dtype=i32) * 97) % N
    pd2 = N + jnp.arange(p2, dtype=i32) % DUMMY
    srcf = jnp.concatenate([src, ps2]).reshape(16, CH2, 128)
    dstf = jnp.concatenate([dst, pd2]).reshape(16, CH2, 128)
    src2 = jnp.stack([srcf, srcf + NPAD])
    dst2 = jnp.stack([dstf, dstf])
    return src1, dst1, src2, dst2


def kernel(x_feature, llmfeatures, W1a, W1b, W2a, W2b, Wt1, Wt2, Wp1, Wp2,
           Wm1, Wm2, edge_index):
    x = jnp.pad(x_feature, ((0, NPAD - N), (0, 0)))
    llm = jnp.pad(llmfeatures, ((0, NPAD - N), (0, 0)))
    src1, dst1, src2, dst2 = _edge_layout(edge_index[0], edge_index[1])

    degp = _sc_deg(dst1)
    dp = degp.reshape(2, NPAD, 1)
    disb, zt = _tc_dis(dp, x)
    z = x
    for _ in range(PPR_ITERS):
        p = _sc_scatter128(zt, src1, dst1)
        z, zt = _tc_ppr(x, zt, p, disb)

    a1t = _tc_mm_scale(z, W1a, disb)
    s1 = _sc_scatter256(a1t, src2, dst2)
    b1t = _tc_fuse_mm(s1, a1t, W1b, disb)
    s2 = _sc_scatter256(b1t, src2, dst2)

    a2t = _tc_mm_scale(llm, W2a, disb)
    s3 = _sc_scatter256(a2t, src2, dst2)
    b2t = _tc_fuse_mm(s3, a2t, W2b, disb)
    s4 = _sc_scatter256(b2t, src2, dst2)

    z1, z3, zz = _tc_final(s2, b1t, s4, b2t, disb)
    return (z3[:N], zz[:N], z1[:N], z1[:N])
